# baseline (speedup 1.0000x reference)
import functools

import jax
import jax.numpy as jnp
from jax import lax
from jax.experimental import pallas as pl
from jax.experimental.pallas import tpu as pltpu
from jax.experimental.pallas import tpu_sc as plsc

_B = 8
_N = 20000
_M = 256
_S = 512
_R = 1.0
_D = 8
_LANES = 16
_G = 4
_BLK = 10
_IDX_CAP = 688


def _box_prep_body(bx_ref, bo_ref, out_ref):
    bx = bx_ref[...]
    o = bo_ref[...]
    c = jnp.cos(o)[..., None]
    s = jnp.sin(o)[..., None]
    z = jnp.zeros_like(c)
    out_ref[...] = jnp.concatenate([bx, c, s] + [z] * 11, axis=-1)


def _box_prep(box_xyz, box_orientations):
    return pl.pallas_call(
        _box_prep_body,
        out_shape=jax.ShapeDtypeStruct((_B, _M, _LANES), jnp.float32),
    )(box_xyz, box_orientations)


def _make_sc_kernel():
    info = plsc.get_sparse_core_info()
    num_cores, num_subcores = info.num_cores, info.num_subcores
    nw = num_cores * num_subcores
    tiles_per_batch = nw // _B
    m_per_tile = _M // tiles_per_batch
    n_chunks = _N // _LANES
    ngroups = m_per_tile // _G

    mesh = plsc.VectorSubcoreMesh(core_axis_name="c", subcore_axis_name="s")

    @functools.partial(
        pl.kernel,
        mesh=mesh,
        compiler_params=pltpu.CompilerParams(needs_layout_passes=False,
                                             use_tc_tiling_on_sc=False),
        out_type=(
            jax.ShapeDtypeStruct((_M * _B * _S * 3,), jnp.float32),
            jax.ShapeDtypeStruct((_B * 5 * _M * _S,), jnp.float32),
        ),
        scratch_types=[
            pltpu.VMEM((_N,), jnp.float32),
            pltpu.VMEM((_N,), jnp.float32),
            pltpu.VMEM((_N,), jnp.float32),
            pltpu.VMEM((m_per_tile * _LANES,), jnp.float32),
            pltpu.VMEM((_G * _IDX_CAP,), jnp.int32),
            pltpu.VMEM((_G * _S,), jnp.int32),
            pltpu.VMEM((_G * _S, _D), jnp.float32),
            pltpu.VMEM((2 * _G * _S * 3,), jnp.float32),
            pltpu.VMEM((2 * _G * 5 * _S,), jnp.float32),
            pltpu.SemaphoreType.DMA,
            pltpu.SemaphoreType.DMA,
        ],
    )
    def sc_group(table_hbm, soa_hbm, boxp_hbm, oxyz_hbm, ofeat_hbm,
                 xs, ys, zs, boxq, idxl, idxg, rows, obx, obf,
                 sem_in, sem_out):
        cid = lax.axis_index("c")
        sid = lax.axis_index("s")
        wid = sid * num_cores + cid
        b = wid // tiles_per_batch
        m0 = (wid % tiles_per_batch) * m_per_tile

        pltpu.sync_copy(soa_hbm.at[pl.ds((b * 3 + 0) * _N, _N)], xs)
        pltpu.sync_copy(soa_hbm.at[pl.ds((b * 3 + 1) * _N, _N)], ys)
        pltpu.sync_copy(soa_hbm.at[pl.ds((b * 3 + 2) * _N, _N)], zs)
        pltpu.sync_copy(
            boxp_hbm.at[pl.ds((b * _M + m0) * _LANES, m_per_tile * _LANES)],
            boxq)

        iota = lax.iota(jnp.int32, _LANES)
        base_g = b * _N

        def g_body(g, carry):
            p = g % 2
            q0 = g * _G
            prms = [boxq[pl.ds((q0 + k) * _LANES, _LANES)]
                    for k in range(_G)]
            cxs = [prm[0] for prm in prms]
            cys = [prm[1] for prm in prms]
            czs = [prm[2] for prm in prms]

            def scan_cond(st):
                i = st[0]
                alive = st[1] < _S
                for c in st[2:]:
                    alive = jnp.logical_or(alive, c < _S)
                return jnp.logical_and(i < n_chunks // _BLK, alive)

            def scan_step(st):
                i = st[0]
                cs = list(st[1:])
                gates = [jnp.broadcast_to(c < _S, (_LANES,)) for c in cs]
                off = i * (_BLK * _LANES)
                for j in range(_BLK):
                    o2 = off + j * _LANES
                    xv = xs[pl.ds(o2, _LANES)]
                    yv = ys[pl.ds(o2, _LANES)]
                    zv = zs[pl.ds(o2, _LANES)]
                    vals = (base_g + o2) + iota
                    mks = []
                    for k in range(_G):
                        dx = xv - cxs[k]
                        dy = yv - cys[k]
                        dz = zv - czs[k]
                        d2 = dx * dx + dy * dy + dz * dz
                        mks.append(jnp.logical_and(d2 < (_R * _R),
                                                   gates[k]))
                    pcs = [plsc.all_reduce_population_count(mk)[0]
                           for mk in mks]
                    for k in range(_G):
                        plsc.store_compressed(
                            idxl.at[pl.ds(k * _IDX_CAP + cs[k], _LANES)],
                            vals, mask=mks[k])
                        cs[k] = cs[k] + pcs[k]
                return (i + jnp.int32(1),) + tuple(cs)

            st = lax.while_loop(
                scan_cond, scan_step,
                (jnp.int32(0),) + (jnp.int32(0),) * _G)
            cnts = st[1:]

            for k in range(_G):
                cnt = cnts[k]
                first = jnp.where(
                    cnt > 0, idxl[pl.ds(k * _IDX_CAP, _LANES)][0], base_g)
                fvec = jnp.broadcast_to(first, (_LANES,))
                for j in range(4):
                    def fill(t, _, j=j, k=k, cnt=cnt, fvec=fvec):
                        pos = j * 128 + t * _LANES
                        v = idxl[pl.ds(k * _IDX_CAP + pos, _LANES)]
                        v = jnp.where((pos + iota) < cnt, v, fvec)
                        idxg[pl.ds(k * _S + pos, _LANES)] = v
                        return 0

                    lax.fori_loop(0, 128 // _LANES, fill, 0)
                    pltpu.async_copy(
                        table_hbm.at[
                            idxg.at[pl.ds(k * _S + j * 128, 128)]],
                        rows.at[pl.ds(k * _S + j * 128, 128)],
                        sem_in)
            pltpu.make_async_copy(
                table_hbm.at[pl.ds(0, _G * _S)], rows, sem_in).wait()

            @pl.when(g > 1)
            def _drain_outs():
                pltpu.make_async_copy(
                    oxyz_hbm.at[pl.ds(0, _G * _S * 3)],
                    obx.at[pl.ds(p * _G * _S * 3, _G * _S * 3)],
                    sem_out).wait()
                pltpu.make_async_copy(
                    ofeat_hbm.at[pl.ds(0, _G * 5 * _S)],
                    obf.at[pl.ds(p * _G * 5 * _S, _G * 5 * _S)],
                    sem_out).wait()

            for k in range(_G):
                prm = prms[k]
                cx = cxs[k]
                cy = cys[k]
                cz = czs[k]
                co = prm[3]
                si = prm[4]
                rbase = k * _S
                xbase = (p * _G + k) * _S * 3
                fbase = (p * _G + k) * 5 * _S

                def transform(t, _, rbase=rbase, xbase=xbase, fbase=fbase,
                              cx=cx, cy=cy, cz=cz, co=co, si=si):
                    rid = t * _LANES + iota
                    col = [
                        plsc.load_gather(
                            rows,
                            [rbase + rid, jnp.full((_LANES,), c,
                                                   jnp.int32)])
                        for c in range(_D)
                    ]
                    inv_r = 1.0 / _R
                    gx = (col[0] - cx) * inv_r
                    gy = (col[1] - cy) * inv_r
                    gz = (col[2] - cz) * inv_r
                    xr = co * gx + si * gy
                    yr = co * gy - si * gx
                    tgt = xbase + rid * 3
                    plsc.store_scatter(obx, [tgt], xr)
                    plsc.store_scatter(obx, [tgt + 1], yr)
                    plsc.store_scatter(obx, [tgt + 2], gz)
                    for c in range(5):
                        obf[pl.ds(fbase + c * _S + t * _LANES,
                                  _LANES)] = col[3 + c]
                    return 0

                lax.fori_loop(0, _S // _LANES, transform, 0)

                m = m0 + q0 + k
                pltpu.async_copy(
                    obx.at[pl.ds(xbase, _S * 3)],
                    oxyz_hbm.at[pl.ds((m * _B + b) * _S * 3, _S * 3)],
                    sem_out)
                for c in range(5):
                    pltpu.async_copy(
                        obf.at[pl.ds(fbase + c * _S, _S)],
                        ofeat_hbm.at[
                            pl.ds(((b * 5 + c) * _M + m) * _S, _S)],
                        sem_out)
            return carry

        lax.fori_loop(0, ngroups, g_body, 0)

        for bank in ((ngroups - 2) % 2, (ngroups - 1) % 2):
            pltpu.make_async_copy(
                oxyz_hbm.at[pl.ds(0, _G * _S * 3)],
                obx.at[pl.ds(bank * _G * _S * 3, _G * _S * 3)],
                sem_out).wait()
            pltpu.make_async_copy(
                ofeat_hbm.at[pl.ds(0, _G * 5 * _S)],
                obf.at[pl.ds(bank * _G * 5 * _S, _G * 5 * _S)],
                sem_out).wait()

    return sc_group


_sc_group = None


def kernel(box_xyz, box_orientations, box_feature, input_point_cloud,
           point_instance_labels, proposal_instance_labels):
    global _sc_group
    if _sc_group is None:
        _sc_group = _make_sc_kernel()

    pc = input_point_cloud.astype(jnp.float32)
    table = jnp.concatenate(
        [pc, point_instance_labels.astype(jnp.float32)[..., None]], axis=-1)
    soa = jnp.transpose(table[..., :3], (0, 2, 1)).reshape(-1)
    table_flat = table.reshape(_B * _N, _D)
    boxp = _box_prep(box_xyz.astype(jnp.float32),
                     box_orientations.astype(jnp.float32)).reshape(-1)

    xyz_flat, feat_flat = _sc_group(table_flat, soa, boxp)
    xyz_out = xyz_flat.reshape(_M, _B, _S, 3)
    feat_out = feat_flat.reshape(_B, 5, _M, _S)
    return xyz_out, feat_out


# speedup vs baseline: 1.3294x; 1.3294x over previous
import functools

import jax
import jax.numpy as jnp
from jax import lax
from jax.experimental import pallas as pl
from jax.experimental.pallas import tpu as pltpu
from jax.experimental.pallas import tpu_sc as plsc

_B = 8
_N = 20000
_M = 256
_S = 512
_R = 1.0
_D = 8
_LANES = 16
_BLK = 50
_IDX_CAP = 1328


def _box_prep_body(bx_ref, bo_ref, out_ref):
    bx = bx_ref[...]
    o = bo_ref[...]
    c = jnp.cos(o)[..., None]
    s = jnp.sin(o)[..., None]
    z = jnp.zeros_like(c)
    out_ref[...] = jnp.concatenate([bx, c, s] + [z] * 11, axis=-1)


def _box_prep(box_xyz, box_orientations):
    return pl.pallas_call(
        _box_prep_body,
        out_shape=jax.ShapeDtypeStruct((_B, _M, _LANES), jnp.float32),
    )(box_xyz, box_orientations)


def _make_sc_kernel():
    info = plsc.get_sparse_core_info()
    num_cores, num_subcores = info.num_cores, info.num_subcores
    nw = num_cores * num_subcores
    tiles_per_batch = nw // _B
    m_per_tile = _M // tiles_per_batch
    n_chunks = _N // _LANES

    mesh = plsc.VectorSubcoreMesh(core_axis_name="c", subcore_axis_name="s")

    @functools.partial(
        pl.kernel,
        mesh=mesh,
        compiler_params=pltpu.CompilerParams(needs_layout_passes=False,
                                             use_tc_tiling_on_sc=False),
        out_type=(
            jax.ShapeDtypeStruct((_M * _B * _S * 3,), jnp.float32),
            jax.ShapeDtypeStruct((_B * 5 * _M * _S,), jnp.float32),
        ),
        scratch_types=[
            pltpu.VMEM((_N,), jnp.float32),
            pltpu.VMEM((_N,), jnp.float32),
            pltpu.VMEM((_N,), jnp.float32),
            pltpu.VMEM((m_per_tile * _LANES,), jnp.float32),
            pltpu.VMEM((_IDX_CAP,), jnp.int32),
            pltpu.VMEM((2 * _S,), jnp.int32),
            pltpu.VMEM((2 * _S, _D), jnp.float32),
            pltpu.VMEM((2 * _S * 3,), jnp.float32),
            pltpu.VMEM((2 * 5 * _S,), jnp.float32),
            pltpu.SemaphoreType.DMA,
            pltpu.SemaphoreType.DMA,
            pltpu.SemaphoreType.DMA,
        ],
    )
    def sc_group(table_hbm, soa_hbm, boxp_hbm, oxyz_hbm, ofeat_hbm,
                 xs, ys, zs, boxq, idxl, idxg, rows, obx, obf,
                 sem_in0, sem_in1, sem_out):
        cid = lax.axis_index("c")
        sid = lax.axis_index("s")
        wid = sid * num_cores + cid
        b = wid // tiles_per_batch
        m0 = (wid % tiles_per_batch) * m_per_tile

        pltpu.sync_copy(soa_hbm.at[pl.ds((b * 3 + 0) * _N, _N)], xs)
        pltpu.sync_copy(soa_hbm.at[pl.ds((b * 3 + 1) * _N, _N)], ys)
        pltpu.sync_copy(soa_hbm.at[pl.ds((b * 3 + 2) * _N, _N)], zs)
        pltpu.sync_copy(
            boxp_hbm.at[pl.ds((b * _M + m0) * _LANES, m_per_tile * _LANES)],
            boxq)

        iota = lax.iota(jnp.int32, _LANES)
        base_g = b * _N

        def q_body(q, carry):
            p = q % 2

            @pl.when(q < m_per_tile)
            def _scan_fire():
                prm = boxq[pl.ds(q * _LANES, _LANES)]
                cx = prm[0]
                cy = prm[1]
                cz = prm[2]
                _scan_query(cx, cy, cz, p)

            @pl.when(q > 1)
            def _drain_outs():
                pltpu.make_async_copy(
                    oxyz_hbm.at[pl.ds(0, _S * 3)],
                    obx.at[pl.ds(p * _S * 3, _S * 3)], sem_out).wait()
                pltpu.make_async_copy(
                    ofeat_hbm.at[pl.ds(0, 5 * _S)],
                    obf.at[pl.ds(p * 5 * _S, 5 * _S)], sem_out).wait()

            @pl.when(q > 0)
            def _finish_prev():
                pp = (q - 1) % 2
                @pl.when(pp == 0)
                def _():
                    pltpu.make_async_copy(
                        table_hbm.at[pl.ds(0, _S)],
                        rows.at[pl.ds(0, _S)], sem_in0).wait()

                @pl.when(pp == 1)
                def _():
                    pltpu.make_async_copy(
                        table_hbm.at[pl.ds(0, _S)],
                        rows.at[pl.ds(_S, _S)], sem_in1).wait()

                _transform_fire(q - 1, pp)

            return carry

        def _scan_query(cx, cy, cz, p):
            def scan_cond(st):
                i, cnt = st
                return jnp.logical_and(i < n_chunks // _BLK, cnt < _S)

            def scan_step(st):
                i, cnt = st
                off = i * (_BLK * _LANES)
                mks = []
                for j in range(_BLK):
                    o2 = off + j * _LANES
                    dx = xs[pl.ds(o2, _LANES)] - cx
                    dy = ys[pl.ds(o2, _LANES)] - cy
                    dz = zs[pl.ds(o2, _LANES)] - cz
                    d2 = dx * dx + dy * dy + dz * dz
                    mks.append(d2 < (_R * _R))
                pcs = [plsc.all_reduce_population_count(mk)[0] for mk in mks]
                for j in range(_BLK):
                    vals = (base_g + off + j * _LANES) + iota
                    plsc.store_compressed(idxl.at[pl.ds(cnt, _LANES)], vals,
                                          mask=mks[j])
                    cnt = cnt + pcs[j]
                return (i + jnp.int32(1), cnt)

            _, cnt = lax.while_loop(
                scan_cond, scan_step, (jnp.int32(0), jnp.int32(0)))

            first = jnp.where(cnt > 0, idxl[pl.ds(0, _LANES)][0], base_g)
            fvec = jnp.broadcast_to(first, (_LANES,))

            def fire(bank, sem):
                for j in range(4):
                    def fill(t, _, j=j):
                        pos = j * 128 + t * _LANES
                        v = idxl[pl.ds(pos, _LANES)]
                        v = jnp.where((pos + iota) < cnt, v, fvec)
                        idxg[pl.ds(bank * _S + pos, _LANES)] = v
                        return 0

                    lax.fori_loop(0, 128 // _LANES, fill, 0)
                    pltpu.async_copy(
                        table_hbm.at[
                            idxg.at[pl.ds(bank * _S + j * 128, 128)]],
                        rows.at[pl.ds(bank * _S + j * 128, 128)],
                        sem)

            @pl.when(p == 0)
            def _():
                fire(0, sem_in0)

            @pl.when(p == 1)
            def _():
                fire(1, sem_in1)

        def _transform_fire(qq, pp):
            prm = boxq[pl.ds(qq * _LANES, _LANES)]
            cx = prm[0]
            cy = prm[1]
            cz = prm[2]
            co = prm[3]
            si = prm[4]
            rbase = pp * _S
            xbase = pp * _S * 3
            fbase = pp * 5 * _S

            def transform(t, _):
                rid = t * _LANES + iota
                col = [
                    plsc.load_gather(
                        rows,
                        [rbase + rid, jnp.full((_LANES,), c, jnp.int32)])
                    for c in range(_D)
                ]
                inv_r = 1.0 / _R
                gx = (col[0] - cx) * inv_r
                gy = (col[1] - cy) * inv_r
                gz = (col[2] - cz) * inv_r
                xr = co * gx + si * gy
                yr = co * gy - si * gx
                tgt = xbase + rid * 3
                plsc.store_scatter(obx, [tgt], xr)
                plsc.store_scatter(obx, [tgt + 1], yr)
                plsc.store_scatter(obx, [tgt + 2], gz)
                for c in range(5):
                    obf[pl.ds(fbase + c * _S + t * _LANES, _LANES)] = \
                        col[3 + c]
                return 0

            lax.fori_loop(0, _S // _LANES, transform, 0)

            m = m0 + qq
            pltpu.async_copy(
                obx.at[pl.ds(xbase, _S * 3)],
                oxyz_hbm.at[pl.ds((m * _B + b) * _S * 3, _S * 3)],
                sem_out)
            for c in range(5):
                pltpu.async_copy(
                    obf.at[pl.ds(fbase + c * _S, _S)],
                    ofeat_hbm.at[pl.ds(((b * 5 + c) * _M + m) * _S, _S)],
                    sem_out)

        lax.fori_loop(0, m_per_tile + 1, q_body, 0)

        lastp = (m_per_tile - 1) % 2
        pltpu.make_async_copy(
            oxyz_hbm.at[pl.ds(0, _S * 3)],
            obx.at[pl.ds(lastp * _S * 3, _S * 3)], sem_out).wait()
        pltpu.make_async_copy(
            ofeat_hbm.at[pl.ds(0, 5 * _S)],
            obf.at[pl.ds(lastp * 5 * _S, 5 * _S)], sem_out).wait()

    return sc_group


_sc_group = None


def kernel(box_xyz, box_orientations, box_feature, input_point_cloud,
           point_instance_labels, proposal_instance_labels):
    global _sc_group
    if _sc_group is None:
        _sc_group = _make_sc_kernel()

    pc = input_point_cloud.astype(jnp.float32)
    table = jnp.concatenate(
        [pc, point_instance_labels.astype(jnp.float32)[..., None]], axis=-1)
    soa = jnp.transpose(table[..., :3], (0, 2, 1)).reshape(-1)
    table_flat = table.reshape(_B * _N, _D)
    boxp = _box_prep(box_xyz.astype(jnp.float32),
                     box_orientations.astype(jnp.float32)).reshape(-1)

    xyz_flat, feat_flat = _sc_group(table_flat, soa, boxp)
    xyz_out = xyz_flat.reshape(_M, _B, _S, 3)
    feat_out = feat_flat.reshape(_B, 5, _M, _S)
    return xyz_out, feat_out


# speedup vs baseline: 1.3309x; 1.0011x over previous
import functools

import jax
import jax.numpy as jnp
from jax import lax
from jax.experimental import pallas as pl
from jax.experimental.pallas import tpu as pltpu
from jax.experimental.pallas import tpu_sc as plsc

_B = 8
_N = 20000
_M = 256
_S = 512
_R = 1.0
_D = 8
_LANES = 16
_BLK = 25
_IDX_CAP = 928


def _box_prep_body(bx_ref, bo_ref, out_ref):
    bx = bx_ref[...]
    o = bo_ref[...]
    c = jnp.cos(o)[..., None]
    s = jnp.sin(o)[..., None]
    z = jnp.zeros_like(c)
    out_ref[...] = jnp.concatenate([bx, c, s] + [z] * 11, axis=-1)


def _box_prep(box_xyz, box_orientations):
    return pl.pallas_call(
        _box_prep_body,
        out_shape=jax.ShapeDtypeStruct((_B, _M, _LANES), jnp.float32),
    )(box_xyz, box_orientations)


def _make_sc_kernel():
    info = plsc.get_sparse_core_info()
    num_cores, num_subcores = info.num_cores, info.num_subcores
    nw = num_cores * num_subcores
    tiles_per_batch = nw // _B
    m_per_tile = _M // tiles_per_batch
    n_chunks = _N // _LANES

    mesh = plsc.VectorSubcoreMesh(core_axis_name="c", subcore_axis_name="s")

    @functools.partial(
        pl.kernel,
        mesh=mesh,
        compiler_params=pltpu.CompilerParams(needs_layout_passes=False,
                                             use_tc_tiling_on_sc=False),
        out_type=(
            jax.ShapeDtypeStruct((_M * _B * _S * 3,), jnp.float32),
            jax.ShapeDtypeStruct((_B * 5 * _M * _S,), jnp.float32),
        ),
        scratch_types=[
            pltpu.VMEM((_N,), jnp.float32),
            pltpu.VMEM((_N,), jnp.float32),
            pltpu.VMEM((_N,), jnp.float32),
            pltpu.VMEM((m_per_tile * _LANES,), jnp.float32),
            pltpu.VMEM((_IDX_CAP,), jnp.int32),
            pltpu.VMEM((2 * _S,), jnp.int32),
            pltpu.VMEM((2 * _S, _D), jnp.float32),
            pltpu.VMEM((2 * _S * 3,), jnp.float32),
            pltpu.VMEM((2 * 5 * _S,), jnp.float32),
            pltpu.SemaphoreType.DMA,
            pltpu.SemaphoreType.DMA,
            pltpu.SemaphoreType.DMA,
        ],
    )
    def sc_group(table_hbm, soa_hbm, boxp_hbm, oxyz_hbm, ofeat_hbm,
                 xs, ys, zs, boxq, idxl, idxg, rows, obx, obf,
                 sem_in0, sem_in1, sem_out):
        cid = lax.axis_index("c")
        sid = lax.axis_index("s")
        wid = sid * num_cores + cid
        b = wid // tiles_per_batch
        m0 = (wid % tiles_per_batch) * m_per_tile

        pltpu.sync_copy(soa_hbm.at[pl.ds((b * 3 + 0) * _N, _N)], xs)
        pltpu.sync_copy(soa_hbm.at[pl.ds((b * 3 + 1) * _N, _N)], ys)
        pltpu.sync_copy(soa_hbm.at[pl.ds((b * 3 + 2) * _N, _N)], zs)
        pltpu.sync_copy(
            boxp_hbm.at[pl.ds((b * _M + m0) * _LANES, m_per_tile * _LANES)],
            boxq)

        iota = lax.iota(jnp.int32, _LANES)
        base_g = b * _N

        def q_body(q, carry):
            p = q % 2

            @pl.when(q < m_per_tile)
            def _scan_fire():
                prm = boxq[pl.ds(q * _LANES, _LANES)]
                cx = prm[0]
                cy = prm[1]
                cz = prm[2]
                _scan_query(cx, cy, cz, p)

            @pl.when(q > 1)
            def _drain_outs():
                pltpu.make_async_copy(
                    oxyz_hbm.at[pl.ds(0, _S * 3)],
                    obx.at[pl.ds(p * _S * 3, _S * 3)], sem_out).wait()
                pltpu.make_async_copy(
                    ofeat_hbm.at[pl.ds(0, 5 * _S)],
                    obf.at[pl.ds(p * 5 * _S, 5 * _S)], sem_out).wait()

            @pl.when(q > 0)
            def _finish_prev():
                pp = (q - 1) % 2
                @pl.when(pp == 0)
                def _():
                    pltpu.make_async_copy(
                        table_hbm.at[pl.ds(0, _S)],
                        rows.at[pl.ds(0, _S)], sem_in0).wait()

                @pl.when(pp == 1)
                def _():
                    pltpu.make_async_copy(
                        table_hbm.at[pl.ds(0, _S)],
                        rows.at[pl.ds(_S, _S)], sem_in1).wait()

                _transform_fire(q - 1, pp)

            return carry

        def _scan_query(cx, cy, cz, p):
            def scan_cond(st):
                i, cnt = st
                return jnp.logical_and(i < n_chunks // _BLK, cnt < _S)

            def scan_step(st):
                i, cnt = st
                off = i * (_BLK * _LANES)
                mks = []
                for j in range(_BLK):
                    o2 = off + j * _LANES
                    dx = xs[pl.ds(o2, _LANES)] - cx
                    dy = ys[pl.ds(o2, _LANES)] - cy
                    dz = zs[pl.ds(o2, _LANES)] - cz
                    d2 = dx * dx + dy * dy + dz * dz
                    mks.append(d2 < (_R * _R))
                pcs = [plsc.all_reduce_population_count(mk)[0] for mk in mks]
                for j in range(_BLK):
                    vals = (base_g + off + j * _LANES) + iota
                    plsc.store_compressed(idxl.at[pl.ds(cnt, _LANES)], vals,
                                          mask=mks[j])
                    cnt = cnt + pcs[j]
                return (i + jnp.int32(1), cnt)

            _, cnt = lax.while_loop(
                scan_cond, scan_step, (jnp.int32(0), jnp.int32(0)))

            first = jnp.where(cnt > 0, idxl[pl.ds(0, _LANES)][0], base_g)
            fvec = jnp.broadcast_to(first, (_LANES,))

            def fire(bank, sem):
                for j in range(4):
                    def fill(t, _, j=j):
                        pos = j * 128 + t * _LANES
                        v = idxl[pl.ds(pos, _LANES)]
                        v = jnp.where((pos + iota) < cnt, v, fvec)
                        idxg[pl.ds(bank * _S + pos, _LANES)] = v
                        return 0

                    lax.fori_loop(0, 128 // _LANES, fill, 0)
                    pltpu.async_copy(
                        table_hbm.at[
                            idxg.at[pl.ds(bank * _S + j * 128, 128)]],
                        rows.at[pl.ds(bank * _S + j * 128, 128)],
                        sem)

            @pl.when(p == 0)
            def _():
                fire(0, sem_in0)

            @pl.when(p == 1)
            def _():
                fire(1, sem_in1)

        def _transform_fire(qq, pp):
            prm = boxq[pl.ds(qq * _LANES, _LANES)]
            cx = prm[0]
            cy = prm[1]
            cz = prm[2]
            co = prm[3]
            si = prm[4]
            rbase = pp * _S
            xbase = pp * _S * 3
            fbase = pp * 5 * _S

            def transform(t, _):
                rid = t * _LANES + iota
                col = [
                    plsc.load_gather(
                        rows,
                        [rbase + rid, jnp.full((_LANES,), c, jnp.int32)])
                    for c in range(_D)
                ]
                inv_r = 1.0 / _R
                gx = (col[0] - cx) * inv_r
                gy = (col[1] - cy) * inv_r
                gz = (col[2] - cz) * inv_r
                xr = co * gx + si * gy
                yr = co * gy - si * gx
                tgt = xbase + rid * 3
                plsc.store_scatter(obx, [tgt], xr)
                plsc.store_scatter(obx, [tgt + 1], yr)
                plsc.store_scatter(obx, [tgt + 2], gz)
                for c in range(5):
                    obf[pl.ds(fbase + c * _S + t * _LANES, _LANES)] = \
                        col[3 + c]
                return 0

            lax.fori_loop(0, _S // _LANES, transform, 0)

            m = m0 + qq
            pltpu.async_copy(
                obx.at[pl.ds(xbase, _S * 3)],
                oxyz_hbm.at[pl.ds((m * _B + b) * _S * 3, _S * 3)],
                sem_out)
            for c in range(5):
                pltpu.async_copy(
                    obf.at[pl.ds(fbase + c * _S, _S)],
                    ofeat_hbm.at[pl.ds(((b * 5 + c) * _M + m) * _S, _S)],
                    sem_out)

        lax.fori_loop(0, m_per_tile + 1, q_body, 0)

        lastp = (m_per_tile - 1) % 2
        pltpu.make_async_copy(
            oxyz_hbm.at[pl.ds(0, _S * 3)],
            obx.at[pl.ds(lastp * _S * 3, _S * 3)], sem_out).wait()
        pltpu.make_async_copy(
            ofeat_hbm.at[pl.ds(0, 5 * _S)],
            obf.at[pl.ds(lastp * 5 * _S, 5 * _S)], sem_out).wait()

    return sc_group


_sc_group = None


def kernel(box_xyz, box_orientations, box_feature, input_point_cloud,
           point_instance_labels, proposal_instance_labels):
    global _sc_group
    if _sc_group is None:
        _sc_group = _make_sc_kernel()

    pc = input_point_cloud.astype(jnp.float32)
    table = jnp.concatenate(
        [pc, point_instance_labels.astype(jnp.float32)[..., None]], axis=-1)
    soa = jnp.transpose(table[..., :3], (0, 2, 1)).reshape(-1)
    table_flat = table.reshape(_B * _N, _D)
    boxp = _box_prep(box_xyz.astype(jnp.float32),
                     box_orientations.astype(jnp.float32)).reshape(-1)

    xyz_flat, feat_flat = _sc_group(table_flat, soa, boxp)
    xyz_out = xyz_flat.reshape(_M, _B, _S, 3)
    feat_out = feat_flat.reshape(_B, 5, _M, _S)
    return xyz_out, feat_out
